# baseline (speedup 1.0000x reference)
import jax
import jax.numpy as jnp
from jax.experimental import pallas as pl

EPS = 1e-05
LOG2E = 1.4426950408889634
CAP = 80.0
N_EXACT = 8


def _body(x_ref, clo_ref, chi_ref, wlo_ref, whi_ref, o_ref):
    x = x_ref[...]
    c_lo = clo_ref[...]
    d = LOG2E / (jnp.abs(x - c_lo) + EPS)
    u_lo = jnp.exp2(jnp.minimum(d, CAP))
    c_hi = chi_ref[...]
    a0 = jnp.exp2(LOG2E / c_hi)
    a1 = jnp.exp2(LOG2E / (c_hi - 1.0))
    ab = jnp.dot(whi_ref[...], jnp.concatenate([a0, a1 - a0], axis=1),
                 preferred_element_type=jnp.float32)
    w10 = jnp.concatenate([wlo_ref[...], ab], axis=1)
    u10 = jnp.concatenate([u_lo, jnp.ones_like(x), x], axis=0)
    vs = jnp.dot(w10, u10,
                 preferred_element_type=jnp.float32)
    embed_dim = vs.shape[0] - 1
    o_ref[...] = vs[:embed_dim, :] * (1.0 / vs[embed_dim:, :])


def kernel(x, emb_weight, centroid):
    batch = x.shape[0]
    num_level, embed_dim = emb_weight.shape
    x_row = x.reshape(1, batch)
    w_aug_t = jnp.concatenate(
        [emb_weight.T, jnp.ones((1, num_level), jnp.float32)], axis=0)
    c_lo = centroid[:N_EXACT]
    c_hi = centroid[N_EXACT:]
    w_lo = w_aug_t[:, :N_EXACT]
    w_hi = w_aug_t[:, N_EXACT:]
    n_hi = num_level - N_EXACT
    block_b = batch // 4
    out_t = pl.pallas_call(
        _body,
        grid=(4,),
        in_specs=[
            pl.BlockSpec((1, block_b), lambda i: (0, i)),
            pl.BlockSpec((N_EXACT, 1), lambda i: (0, 0)),
            pl.BlockSpec((n_hi, 1), lambda i: (0, 0)),
            pl.BlockSpec((embed_dim + 1, N_EXACT), lambda i: (0, 0)),
            pl.BlockSpec((embed_dim + 1, n_hi), lambda i: (0, 0)),
        ],
        out_specs=pl.BlockSpec((embed_dim, block_b), lambda i: (0, i)),
        out_shape=jax.ShapeDtypeStruct((embed_dim, batch), jnp.float32),
    )(x_row, c_lo, c_hi, w_lo, w_hi)
    return out_t.T


# speedup vs baseline: 1.3290x; 1.3290x over previous
import jax
import jax.numpy as jnp
from jax.experimental import pallas as pl

EPS = 1e-05
LOG2E = 1.4426950408889634
CAP = 80.0
BLOCK_B = 16384


def _body(x_ref, c_ref, wt_ref, o_ref):
    x = x_ref[...]
    c = c_ref[...]
    d = LOG2E / (jnp.abs(x - c) + EPS)
    u = jnp.exp2(jnp.minimum(d, CAP))
    vs = jnp.dot(wt_ref[...], u, preferred_element_type=jnp.float32)
    embed_dim = vs.shape[0] - 1
    o_ref[...] = vs[:embed_dim, :] * (1.0 / vs[embed_dim:, :])


def kernel(x, emb_weight, centroid):
    batch = x.shape[0]
    num_level, embed_dim = emb_weight.shape
    x_row = x.reshape(1, batch)
    w_aug_t = jnp.concatenate(
        [emb_weight.T, jnp.ones((1, num_level), jnp.float32)], axis=0)
    grid = batch // BLOCK_B
    out_t = pl.pallas_call(
        _body,
        grid=(grid,),
        in_specs=[
            pl.BlockSpec((1, BLOCK_B), lambda i: (0, i)),
            pl.BlockSpec((num_level, 1), lambda i: (0, 0)),
            pl.BlockSpec((embed_dim + 1, num_level), lambda i: (0, 0)),
        ],
        out_specs=pl.BlockSpec((embed_dim, BLOCK_B), lambda i: (0, i)),
        out_shape=jax.ShapeDtypeStruct((embed_dim, batch), jnp.float32),
    )(x_row, centroid, w_aug_t)
    return out_t.T
